# baseline (speedup 1.0000x reference)
import jax
import jax.numpy as jnp
from jax.experimental import pallas as pl
from jax.experimental.pallas import tpu as pltpu

HID = 1024
INTER = 4096
MINTER = 512
NE = 16

TT = 512
SH_IT = 1024
NSH = INTER // SH_IT
NSTEP = NSH + NE


def _moe_body(x_ref, wg_ref, ws1_ref, ws2_ref, we1_ref, we2_ref,
              out_ref, logits_ref, comb_ref):
    s = pl.program_id(1)

    @pl.when(s == 0)
    def _router():
        x = x_ref[...]
        logits = jax.lax.dot_general(
            x, wg_ref[...], (((1,), (1,)), ((), ())),
            preferred_element_type=jnp.float32)
        logits_ref[...] = logits
        lane = jax.lax.broadcasted_iota(jnp.int32, (TT, NE), 1)
        m1 = jnp.max(logits, axis=1, keepdims=True)
        i1 = jnp.min(jnp.where(logits >= m1, lane, NE), axis=1, keepdims=True)
        masked = jnp.where(lane == i1, -jnp.inf, logits)
        m2 = jnp.max(masked, axis=1, keepdims=True)
        i2 = jnp.min(jnp.where(masked >= m2, lane, NE), axis=1, keepdims=True)
        w1 = 1.0 / (1.0 + jnp.exp(m2 - m1))
        w2 = 1.0 - w1
        comb_ref[...] = (jnp.where(lane == i1, w1, 0.0)
                         + jnp.where(lane == i2, w2, 0.0))
        out_ref[...] = jnp.zeros_like(out_ref)

    @pl.when(s < NSH)
    def _shared():
        x = x_ref[...].astype(jnp.bfloat16)
        h = jax.lax.dot_general(
            x, ws1_ref[...].astype(jnp.bfloat16), (((1,), (1,)), ((), ())),
            preferred_element_type=jnp.float32)
        h = h * jax.nn.sigmoid(h)
        out_ref[...] += jax.lax.dot_general(
            h.astype(jnp.bfloat16), ws2_ref[...].astype(jnp.bfloat16),
            (((1,), (1,)), ((), ())),
            preferred_element_type=jnp.float32)

    @pl.when(s >= NSH)
    def _expert():
        e = s - NSH
        x = x_ref[...].astype(jnp.bfloat16)
        h = jax.lax.dot_general(
            x, we1_ref[0].astype(jnp.bfloat16), (((1,), (1,)), ((), ())),
            preferred_element_type=jnp.float32)
        h = h * jax.nn.sigmoid(h)
        lane = jax.lax.broadcasted_iota(jnp.int32, (TT, NE), 1)
        c = jnp.sum(jnp.where(lane == e, comb_ref[...], 0.0),
                    axis=1, keepdims=True)
        h = h * c
        out_ref[...] += jax.lax.dot_general(
            h.astype(jnp.bfloat16), we2_ref[0].astype(jnp.bfloat16),
            (((1,), (1,)), ((), ())),
            preferred_element_type=jnp.float32)


def kernel(hidden_states, Ws1, Ws2, Wg, We1, We2):
    B, S, H = hidden_states.shape
    T = B * S
    x = hidden_states.reshape(T, H)
    grid = (T // TT, NSTEP)

    out, logits = pl.pallas_call(
        _moe_body,
        grid=grid,
        in_specs=[
            pl.BlockSpec((TT, HID), lambda t, s: (t, 0)),
            pl.BlockSpec((NE, HID), lambda t, s: (0, 0)),
            pl.BlockSpec((SH_IT, HID), lambda t, s: (jnp.minimum(s, NSH - 1), 0)),
            pl.BlockSpec((HID, SH_IT), lambda t, s: (0, jnp.minimum(s, NSH - 1))),
            pl.BlockSpec((1, MINTER, HID), lambda t, s: (jnp.maximum(s - NSH, 0), 0, 0)),
            pl.BlockSpec((1, HID, MINTER), lambda t, s: (jnp.maximum(s - NSH, 0), 0, 0)),
        ],
        out_specs=[
            pl.BlockSpec((TT, HID), lambda t, s: (t, 0)),
            pl.BlockSpec((TT, NE), lambda t, s: (t, 0)),
        ],
        out_shape=[
            jax.ShapeDtypeStruct((T, HID), jnp.float32),
            jax.ShapeDtypeStruct((T, NE), jnp.float32),
        ],
        scratch_shapes=[pltpu.VMEM((TT, NE), jnp.float32)],
        compiler_params=pltpu.CompilerParams(
            dimension_semantics=("parallel", "arbitrary")),
    )(x, Wg, Ws1, Ws2, We1, We2)

    return out.reshape(B, S, H), logits.reshape(B, S, NE)


# speedup vs baseline: 1.1093x; 1.1093x over previous
import functools
import jax
import jax.numpy as jnp
from jax import lax
from jax.experimental import pallas as pl
from jax.experimental.pallas import tpu as pltpu
from jax.experimental.pallas import tpu_sc as plsc

HID = 1024
INTER = 4096
MINTER = 512
NE = 16

TT = 512
NCH = 8
T = NCH * TT
BLK = 256
NBLK = 48
NS = NBLK * BLK
SH_IT = 1024
NSH = INTER // SH_IT
BEPAD = 128

NW = 32
TPW = T // NW
RG = 32
NG = TPW // RG
RG2 = 16
NG2 = TPW // RG2


def _take16(vec, idx):
    dnums = lax.GatherDimensionNumbers(
        offset_dims=(), collapsed_slice_dims=(0,), start_index_map=(0,))
    return lax.gather(vec, idx[:, None], dnums, (1,),
                      mode=lax.GatherScatterMode.PROMISE_IN_BOUNDS)


def _router_body(x_ref, wg_ref, logits_ref, i1_ref, i2_ref, w1_ref, w2_ref,
                 r1_ref, r2_ref, cb_ref, eb_ref, be_ref, fi_ref, cnt_ref):
    s = pl.program_id(0)

    @pl.when(s == 0)
    def _init():
        cnt_ref[...] = jnp.zeros_like(cnt_ref)

    @pl.when(s < NCH)
    def _chunk():
        x = x_ref[...]
        logits = lax.dot_general(x, wg_ref[...], (((1,), (1,)), ((), ())),
                                 preferred_element_type=jnp.float32)
        logits_ref[...] = logits
        lane = lax.broadcasted_iota(jnp.int32, (TT, NE), 1)
        m1 = jnp.max(logits, axis=1, keepdims=True)
        i1 = jnp.min(jnp.where(logits >= m1, lane, NE), axis=1, keepdims=True)
        masked = jnp.where(lane == i1, -jnp.inf, logits)
        m2 = jnp.max(masked, axis=1, keepdims=True)
        i2 = jnp.min(jnp.where(masked >= m2, lane, NE), axis=1, keepdims=True)
        w1 = 1.0 / (1.0 + jnp.exp(m2 - m1))
        oh = ((lane == i1) | (lane == i2)).astype(jnp.float32)
        trow = lax.broadcasted_iota(jnp.int32, (TT, TT), 0)
        tcol = lax.broadcasted_iota(jnp.int32, (TT, TT), 1)
        ls = (tcol < trow).astype(jnp.float32)
        rmat = lax.dot_general(ls, oh, (((1,), (0,)), ((), ())),
                               preferred_element_type=jnp.float32)
        r1 = jnp.sum(jnp.where(lane == i1, rmat, 0.0), axis=1, keepdims=True)
        r2 = jnp.sum(jnp.where(lane == i2, rmat, 0.0), axis=1, keepdims=True)
        i1_ref[...] = i1
        i2_ref[...] = i2
        w1_ref[...] = w1
        w2_ref[...] = 1.0 - w1
        r1_ref[...] = r1.astype(jnp.int32)
        r2_ref[...] = r2.astype(jnp.int32)
        cb_ref[...] = cnt_ref[...].astype(jnp.int32)[None]
        ones_col = jnp.ones((TT, 1), jnp.float32)
        ccol = lax.dot_general(oh, ones_col, (((0,), (0,)), ((), ())),
                               preferred_element_type=jnp.float32)
        cnt_ref[...] += ccol

    @pl.when(s == NCH)
    def _bases():
        total = cnt_ref[...]
        padded = jnp.ceil(total / BLK) * BLK
        erow = lax.broadcasted_iota(jnp.int32, (NE, NE), 0)
        ecol = lax.broadcasted_iota(jnp.int32, (NE, NE), 1)
        lst = (ecol < erow).astype(jnp.float32)
        ebase = lax.dot_general(lst, padded, (((1,), (0,)), ((), ())),
                                preferred_element_type=jnp.float32)
        eb_ref[...] = ebase.astype(jnp.int32)
        tot = jnp.sum(padded)
        bs = (lax.broadcasted_iota(jnp.int32, (1, BEPAD), 1)
              .astype(jnp.float32) * BLK)
        cmp = (ebase <= bs).astype(jnp.float32)
        cnte = jnp.sum(cmp, axis=0, keepdims=True)
        bi = lax.broadcasted_iota(jnp.int32, (1, BEPAD), 1)
        bev = jnp.where(bs < tot, cnte.astype(jnp.int32) - 1, -1)
        nact = (tot / BLK).astype(jnp.int32)
        fiv = jnp.where(bs < tot, jnp.minimum(bi, NBLK - 1), nact - 1)
        be_ref[...] = jnp.reshape(bev, (BEPAD,))
        fi_ref[...] = jnp.reshape(fiv, (BEPAD,))


def _router_call(x, Wg):
    outs = pl.pallas_call(
        _router_body,
        grid=(NCH + 1,),
        in_specs=[
            pl.BlockSpec((TT, HID), lambda s: (jnp.minimum(s, NCH - 1), 0)),
            pl.BlockSpec((NE, HID), lambda s: (0, 0)),
        ],
        out_specs=[
            pl.BlockSpec((TT, NE), lambda s: (jnp.minimum(s, NCH - 1), 0)),
            pl.BlockSpec((TT, 1), lambda s: (jnp.minimum(s, NCH - 1), 0)),
            pl.BlockSpec((TT, 1), lambda s: (jnp.minimum(s, NCH - 1), 0)),
            pl.BlockSpec((TT, 1), lambda s: (jnp.minimum(s, NCH - 1), 0)),
            pl.BlockSpec((TT, 1), lambda s: (jnp.minimum(s, NCH - 1), 0)),
            pl.BlockSpec((TT, 1), lambda s: (jnp.minimum(s, NCH - 1), 0)),
            pl.BlockSpec((TT, 1), lambda s: (jnp.minimum(s, NCH - 1), 0)),
            pl.BlockSpec((1, NE, 1), lambda s: (jnp.minimum(s, NCH - 1), 0, 0)),
            pl.BlockSpec((NE, 1), lambda s: (0, 0)),
            pl.BlockSpec((BEPAD,), lambda s: (0,)),
            pl.BlockSpec((BEPAD,), lambda s: (0,)),
        ],
        out_shape=[
            jax.ShapeDtypeStruct((T, NE), jnp.float32),
            jax.ShapeDtypeStruct((T, 1), jnp.int32),
            jax.ShapeDtypeStruct((T, 1), jnp.int32),
            jax.ShapeDtypeStruct((T, 1), jnp.float32),
            jax.ShapeDtypeStruct((T, 1), jnp.float32),
            jax.ShapeDtypeStruct((T, 1), jnp.int32),
            jax.ShapeDtypeStruct((T, 1), jnp.int32),
            jax.ShapeDtypeStruct((NCH, NE, 1), jnp.int32),
            jax.ShapeDtypeStruct((NE, 1), jnp.int32),
            jax.ShapeDtypeStruct((BEPAD,), jnp.int32),
            jax.ShapeDtypeStruct((BEPAD,), jnp.int32),
        ],
        scratch_shapes=[pltpu.VMEM((NE, 1), jnp.float32)],
        compiler_params=pltpu.CompilerParams(
            dimension_semantics=("arbitrary",)),
    )(x, Wg)
    return outs


def _make_dispatch():
    mesh = plsc.VectorSubcoreMesh(core_axis_name="c", subcore_axis_name="s")

    @functools.partial(
        pl.kernel,
        out_type=[
            jax.ShapeDtypeStruct((NS, HID), jnp.float32),
            jax.ShapeDtypeStruct((T,), jnp.int32),
            jax.ShapeDtypeStruct((T,), jnp.int32),
        ],
        mesh=mesh,
        scratch_types=[
            pltpu.VMEM((TPW,), jnp.int32),
            pltpu.VMEM((TPW,), jnp.int32),
            pltpu.VMEM((NE,), jnp.int32),
            pltpu.VMEM((NE,), jnp.int32),
            pltpu.VMEM((2 * NG, RG), jnp.int32),
            pltpu.VMEM((TPW,), jnp.int32),
            pltpu.VMEM((RG, HID), jnp.float32),
            pltpu.SemaphoreType.DMA,
        ],
    )
    def dispatch(x_hbm, i1_hbm, i2_hbm, r1_hbm, r2_hbm, eb_hbm, cb_hbm,
                 xs_hbm, inv0_hbm, inv1_hbm,
                 iv, rv, ebv, cbv, slots, linv, rows, sem):
        cc = lax.axis_index("c")
        ss = lax.axis_index("s")
        wid = ss * 2 + cc
        base = wid * TPW
        chunk = base // TT
        pltpu.sync_copy(eb_hbm, ebv)
        pltpu.sync_copy(cb_hbm.at[pl.ds(chunk * NE, NE)], cbv)
        cbt = ebv[...] + cbv[...]
        for k in range(2):
            ih = i1_hbm if k == 0 else i2_hbm
            rh = r1_hbm if k == 0 else r2_hbm
            invh = inv0_hbm if k == 0 else inv1_hbm
            pltpu.sync_copy(ih.at[pl.ds(base, TPW)], iv)
            pltpu.sync_copy(rh.at[pl.ds(base, TPW)], rv)
            for v in range(TPW // 16):
                ids = iv[pl.ds(v * 16, 16)]
                bases = _take16(cbt, ids)
                slot = bases + rv[pl.ds(v * 16, 16)]
                g = (v * 16) // RG
                off = (v * 16) % RG
                slots[k * NG + g, pl.ds(off, 16)] = slot
                linv[pl.ds(v * 16, 16)] = slot
            pltpu.sync_copy(linv, invh.at[pl.ds(base, TPW)])
        for g in range(NG):
            pltpu.sync_copy(x_hbm.at[pl.ds(base + g * RG, RG)], rows)
            for k in range(2):
                pltpu.async_copy(rows, xs_hbm.at[slots.at[k * NG + g]],
                                 sem).wait()

    return dispatch


def _expert_body(se_ref, fi_ref, xs_ref, we1_ref, we2_ref, y_ref):
    b = pl.program_id(0)

    @pl.when(se_ref[b] >= 0)
    def _go():
        x = xs_ref[...]
        h = lax.dot_general(x, we1_ref[0],
                            (((1,), (1,)), ((), ())),
                            preferred_element_type=jnp.float32)
        h = h * jax.nn.sigmoid(h)
        y_ref[...] = lax.dot_general(h,
                                     we2_ref[0],
                                     (((1,), (1,)), ((), ())),
                                     preferred_element_type=jnp.float32)


def _expert_call(blk_e, fetch_i, xs, We1, We2):
    grid_spec = pltpu.PrefetchScalarGridSpec(
        num_scalar_prefetch=2,
        grid=(NBLK,),
        in_specs=[
            pl.BlockSpec((BLK, HID), lambda b, se, fi: (fi[b], 0)),
            pl.BlockSpec((1, MINTER, HID),
                         lambda b, se, fi: (se[fi[b]], 0, 0)),
            pl.BlockSpec((1, HID, MINTER),
                         lambda b, se, fi: (se[fi[b]], 0, 0)),
        ],
        out_specs=pl.BlockSpec((BLK, HID), lambda b, se, fi: (b, 0)),
    )
    return pl.pallas_call(
        _expert_body,
        grid_spec=grid_spec,
        out_shape=jax.ShapeDtypeStruct((NS, HID), jnp.float32),
        compiler_params=pltpu.CompilerParams(
            dimension_semantics=("arbitrary",)),
    )(blk_e, fetch_i, xs, We1, We2)


def _shared_body(x_ref, ws1_ref, ws2_ref, o_ref):
    s = pl.program_id(1)

    @pl.when(s == 0)
    def _z():
        o_ref[...] = jnp.zeros_like(o_ref)

    x = x_ref[...]
    h = lax.dot_general(x, ws1_ref[...],
                        (((1,), (1,)), ((), ())),
                        preferred_element_type=jnp.float32)
    h = h * jax.nn.sigmoid(h)
    o_ref[...] += lax.dot_general(h,
                                  ws2_ref[...],
                                  (((1,), (1,)), ((), ())),
                                  preferred_element_type=jnp.float32)


def _shared_call(x, Ws1, Ws2):
    return pl.pallas_call(
        _shared_body,
        grid=(NCH, NSH),
        in_specs=[
            pl.BlockSpec((TT, HID), lambda t, s: (t, 0)),
            pl.BlockSpec((SH_IT, HID), lambda t, s: (s, 0)),
            pl.BlockSpec((HID, SH_IT), lambda t, s: (0, s)),
        ],
        out_specs=pl.BlockSpec((TT, HID), lambda t, s: (t, 0)),
        out_shape=jax.ShapeDtypeStruct((T, HID), jnp.float32),
        compiler_params=pltpu.CompilerParams(
            dimension_semantics=("parallel", "arbitrary")),
    )(x, Ws1, Ws2)


def _make_combine():
    mesh = plsc.VectorSubcoreMesh(core_axis_name="c", subcore_axis_name="s")

    @functools.partial(
        pl.kernel,
        out_type=jax.ShapeDtypeStruct((T, HID), jnp.float32),
        mesh=mesh,
        scratch_types=[
            pltpu.VMEM((TPW,), jnp.int32),
            pltpu.VMEM((TPW,), jnp.int32),
            pltpu.VMEM((TPW,), jnp.float32),
            pltpu.VMEM((TPW,), jnp.float32),
            pltpu.VMEM((RG2, HID), jnp.float32),
            pltpu.VMEM((RG2, HID), jnp.float32),
            pltpu.VMEM((RG2, HID), jnp.float32),
            pltpu.SemaphoreType.DMA,
        ],
    )
    def combine(sh_hbm, y_hbm, inv0_hbm, inv1_hbm, w1_hbm, w2_hbm,
                out_hbm, idx0, idx1, w1v, w2v, acc, y0, y1, sem):
        cc = lax.axis_index("c")
        ss = lax.axis_index("s")
        wid = ss * 2 + cc
        base = wid * TPW
        pltpu.sync_copy(inv0_hbm.at[pl.ds(base, TPW)], idx0)
        pltpu.sync_copy(inv1_hbm.at[pl.ds(base, TPW)], idx1)
        pltpu.sync_copy(w1_hbm.at[pl.ds(base, TPW)], w1v)
        pltpu.sync_copy(w2_hbm.at[pl.ds(base, TPW)], w2v)
        for g in range(NG2):
            pltpu.sync_copy(sh_hbm.at[pl.ds(base + g * RG2, RG2)], acc)
            pltpu.async_copy(y_hbm.at[idx0.at[pl.ds(g * RG2, RG2)]],
                             y0, sem).wait()
            pltpu.async_copy(y_hbm.at[idx1.at[pl.ds(g * RG2, RG2)]],
                             y1, sem).wait()
            w1g = w1v[pl.ds(g * RG2, 16)]
            w2g = w2v[pl.ds(g * RG2, 16)]
            for i in range(RG2):
                sel = jnp.full((16,), i, jnp.int32)
                wa = _take16(w1g, sel)
                wb = _take16(w2g, sel)

                def body(ci, carry):
                    sl = pl.ds(ci * 16, 16)
                    acc[i, sl] = acc[i, sl] + wa * y0[i, sl] + wb * y1[i, sl]
                    return carry

                lax.fori_loop(0, HID // 16, body, 0)
            pltpu.sync_copy(acc, out_hbm.at[pl.ds(base + g * RG2, RG2)])

    return combine


def kernel(hidden_states, Ws1, Ws2, Wg, We1, We2):
    B, S, H = hidden_states.shape
    x = hidden_states.reshape(T, H)

    (logits, i1, i2, w1, w2, r1, r2, cb, eb, be, fi) = _router_call(x, Wg)

    dispatch = _make_dispatch()
    xs, inv0, inv1 = dispatch(x, i1.reshape(T), i2.reshape(T),
                              r1.reshape(T), r2.reshape(T),
                              eb.reshape(NE), cb.reshape(NCH * NE))

    y = _expert_call(be, fi, xs, We1, We2)
    sh = _shared_call(x, Ws1, Ws2)

    combine = _make_combine()
    out = combine(sh, y, inv0, inv1, w1.reshape(T), w2.reshape(T))

    return out.reshape(B, S, H), logits.reshape(B, S, NE)


# speedup vs baseline: 1.2295x; 1.1083x over previous
import functools
import jax
import jax.numpy as jnp
from jax import lax
from jax.experimental import pallas as pl
from jax.experimental.pallas import tpu as pltpu
from jax.experimental.pallas import tpu_sc as plsc

HID = 1024
INTER = 4096
MINTER = 512
NE = 16

TT = 512
NCH = 8
T = NCH * TT
BLK = 256
NBLK = 48
NS = NBLK * BLK
SH_IT = 1024
NSH = INTER // SH_IT
BEPAD = 128

NW = 32
TPW = T // NW
RG = 32
NG = TPW // RG
RG2 = 16
NG2 = TPW // RG2


def _take16(vec, idx):
    dnums = lax.GatherDimensionNumbers(
        offset_dims=(), collapsed_slice_dims=(0,), start_index_map=(0,))
    return lax.gather(vec, idx[:, None], dnums, (1,),
                      mode=lax.GatherScatterMode.PROMISE_IN_BOUNDS)


def _router_body(x_ref, wg_ref, logits_ref, i1_ref, i2_ref, w1_ref, w2_ref,
                 r1_ref, r2_ref, cb_ref, eb_ref, be_ref, fi_ref, cnt_ref):
    s = pl.program_id(0)

    @pl.when(s == 0)
    def _init():
        cnt_ref[...] = jnp.zeros_like(cnt_ref)

    @pl.when(s < NCH)
    def _chunk():
        x = x_ref[...]
        logits = lax.dot_general(x, wg_ref[...], (((1,), (1,)), ((), ())),
                                 preferred_element_type=jnp.float32)
        logits_ref[...] = logits
        lane = lax.broadcasted_iota(jnp.int32, (TT, NE), 1)
        m1 = jnp.max(logits, axis=1, keepdims=True)
        i1 = jnp.min(jnp.where(logits >= m1, lane, NE), axis=1, keepdims=True)
        masked = jnp.where(lane == i1, -jnp.inf, logits)
        m2 = jnp.max(masked, axis=1, keepdims=True)
        i2 = jnp.min(jnp.where(masked >= m2, lane, NE), axis=1, keepdims=True)
        w1 = 1.0 / (1.0 + jnp.exp(m2 - m1))
        oh = ((lane == i1) | (lane == i2)).astype(jnp.float32)
        trow = lax.broadcasted_iota(jnp.int32, (TT, TT), 0)
        tcol = lax.broadcasted_iota(jnp.int32, (TT, TT), 1)
        ls = (tcol < trow).astype(jnp.float32)
        rmat = lax.dot_general(ls, oh, (((1,), (0,)), ((), ())),
                               preferred_element_type=jnp.float32)
        r1 = jnp.sum(jnp.where(lane == i1, rmat, 0.0), axis=1, keepdims=True)
        r2 = jnp.sum(jnp.where(lane == i2, rmat, 0.0), axis=1, keepdims=True)
        i1_ref[...] = i1
        i2_ref[...] = i2
        w1_ref[...] = w1
        w2_ref[...] = 1.0 - w1
        r1_ref[...] = r1.astype(jnp.int32)
        r2_ref[...] = r2.astype(jnp.int32)
        cb_ref[...] = cnt_ref[...].astype(jnp.int32)[None]
        ones_col = jnp.ones((TT, 1), jnp.float32)
        ccol = lax.dot_general(oh, ones_col, (((0,), (0,)), ((), ())),
                               preferred_element_type=jnp.float32)
        cnt_ref[...] += ccol

    @pl.when(s == NCH)
    def _bases():
        total = cnt_ref[...]
        padded = jnp.ceil(total / BLK) * BLK
        erow = lax.broadcasted_iota(jnp.int32, (NE, NE), 0)
        ecol = lax.broadcasted_iota(jnp.int32, (NE, NE), 1)
        lst = (ecol < erow).astype(jnp.float32)
        ebase = lax.dot_general(lst, padded, (((1,), (0,)), ((), ())),
                                preferred_element_type=jnp.float32)
        eb_ref[...] = ebase.astype(jnp.int32)
        tot = jnp.sum(padded)
        bs = (lax.broadcasted_iota(jnp.int32, (1, BEPAD), 1)
              .astype(jnp.float32) * BLK)
        cmp = (ebase <= bs).astype(jnp.float32)
        cnte = jnp.sum(cmp, axis=0, keepdims=True)
        bi = lax.broadcasted_iota(jnp.int32, (1, BEPAD), 1)
        bev = jnp.where(bs < tot, cnte.astype(jnp.int32) - 1, -1)
        nact = (tot / BLK).astype(jnp.int32)
        fiv = jnp.where(bs < tot, jnp.minimum(bi, NBLK - 1), nact - 1)
        be_ref[...] = jnp.reshape(bev, (BEPAD,))
        fi_ref[...] = jnp.reshape(fiv, (BEPAD,))


def _router_call(x, Wg):
    outs = pl.pallas_call(
        _router_body,
        grid=(NCH + 1,),
        in_specs=[
            pl.BlockSpec((TT, HID), lambda s: (jnp.minimum(s, NCH - 1), 0)),
            pl.BlockSpec((NE, HID), lambda s: (0, 0)),
        ],
        out_specs=[
            pl.BlockSpec((TT, NE), lambda s: (jnp.minimum(s, NCH - 1), 0)),
            pl.BlockSpec((TT, 1), lambda s: (jnp.minimum(s, NCH - 1), 0)),
            pl.BlockSpec((TT, 1), lambda s: (jnp.minimum(s, NCH - 1), 0)),
            pl.BlockSpec((TT, 1), lambda s: (jnp.minimum(s, NCH - 1), 0)),
            pl.BlockSpec((TT, 1), lambda s: (jnp.minimum(s, NCH - 1), 0)),
            pl.BlockSpec((TT, 1), lambda s: (jnp.minimum(s, NCH - 1), 0)),
            pl.BlockSpec((TT, 1), lambda s: (jnp.minimum(s, NCH - 1), 0)),
            pl.BlockSpec((1, NE, 1), lambda s: (jnp.minimum(s, NCH - 1), 0, 0)),
            pl.BlockSpec((NE, 1), lambda s: (0, 0)),
            pl.BlockSpec((BEPAD,), lambda s: (0,)),
            pl.BlockSpec((BEPAD,), lambda s: (0,)),
        ],
        out_shape=[
            jax.ShapeDtypeStruct((T, NE), jnp.float32),
            jax.ShapeDtypeStruct((T, 1), jnp.int32),
            jax.ShapeDtypeStruct((T, 1), jnp.int32),
            jax.ShapeDtypeStruct((T, 1), jnp.float32),
            jax.ShapeDtypeStruct((T, 1), jnp.float32),
            jax.ShapeDtypeStruct((T, 1), jnp.int32),
            jax.ShapeDtypeStruct((T, 1), jnp.int32),
            jax.ShapeDtypeStruct((NCH, NE, 1), jnp.int32),
            jax.ShapeDtypeStruct((NE, 1), jnp.int32),
            jax.ShapeDtypeStruct((BEPAD,), jnp.int32),
            jax.ShapeDtypeStruct((BEPAD,), jnp.int32),
        ],
        scratch_shapes=[pltpu.VMEM((NE, 1), jnp.float32)],
        compiler_params=pltpu.CompilerParams(
            dimension_semantics=("arbitrary",)),
    )(x, Wg)
    return outs


def _make_dispatch():
    mesh = plsc.VectorSubcoreMesh(core_axis_name="c", subcore_axis_name="s")

    @functools.partial(
        pl.kernel,
        out_type=[
            jax.ShapeDtypeStruct((NS, HID), jnp.float32),
            jax.ShapeDtypeStruct((T,), jnp.int32),
            jax.ShapeDtypeStruct((T,), jnp.int32),
        ],
        mesh=mesh,
        scratch_types=[
            pltpu.VMEM((TPW,), jnp.int32),
            pltpu.VMEM((TPW,), jnp.int32),
            pltpu.VMEM((NE,), jnp.int32),
            pltpu.VMEM((NE,), jnp.int32),
            pltpu.VMEM((2 * NG, RG), jnp.int32),
            pltpu.VMEM((TPW,), jnp.int32),
            pltpu.VMEM((RG, HID), jnp.float32),
            pltpu.SemaphoreType.DMA,
        ],
    )
    def dispatch(x_hbm, i1_hbm, i2_hbm, r1_hbm, r2_hbm, eb_hbm, cb_hbm,
                 xs_hbm, inv0_hbm, inv1_hbm,
                 iv, rv, ebv, cbv, slots, linv, rows, sem):
        cc = lax.axis_index("c")
        ss = lax.axis_index("s")
        wid = ss * 2 + cc
        base = wid * TPW
        chunk = base // TT
        pltpu.sync_copy(eb_hbm, ebv)
        pltpu.sync_copy(cb_hbm.at[pl.ds(chunk * NE, NE)], cbv)
        cbt = ebv[...] + cbv[...]
        for k in range(2):
            ih = i1_hbm if k == 0 else i2_hbm
            rh = r1_hbm if k == 0 else r2_hbm
            invh = inv0_hbm if k == 0 else inv1_hbm
            pltpu.sync_copy(ih.at[pl.ds(base, TPW)], iv)
            pltpu.sync_copy(rh.at[pl.ds(base, TPW)], rv)
            for v in range(TPW // 16):
                ids = iv[pl.ds(v * 16, 16)]
                bases = _take16(cbt, ids)
                slot = bases + rv[pl.ds(v * 16, 16)]
                g = (v * 16) // RG
                off = (v * 16) % RG
                slots[k * NG + g, pl.ds(off, 16)] = slot
                linv[pl.ds(v * 16, 16)] = slot
            pltpu.sync_copy(linv, invh.at[pl.ds(base, TPW)])
        for g in range(NG):
            pltpu.sync_copy(x_hbm.at[pl.ds(base + g * RG, RG)], rows)
            d0 = pltpu.async_copy(rows, xs_hbm.at[slots.at[g]], sem)
            d1 = pltpu.async_copy(rows, xs_hbm.at[slots.at[NG + g]], sem)
            d0.wait()
            d1.wait()

    return dispatch


def _expert_body(se_ref, fi_ref, xs_ref, we1_ref, we2_ref, y_ref):
    b = pl.program_id(0)

    @pl.when(se_ref[b] >= 0)
    def _go():
        x = xs_ref[...]
        h = lax.dot_general(x, we1_ref[0],
                            (((1,), (1,)), ((), ())),
                            preferred_element_type=jnp.float32)
        h = h * jax.nn.sigmoid(h)
        y_ref[...] = lax.dot_general(h,
                                     we2_ref[0],
                                     (((1,), (1,)), ((), ())),
                                     preferred_element_type=jnp.float32)


def _expert_call(blk_e, fetch_i, xs, We1, We2):
    grid_spec = pltpu.PrefetchScalarGridSpec(
        num_scalar_prefetch=2,
        grid=(NBLK,),
        in_specs=[
            pl.BlockSpec((BLK, HID), lambda b, se, fi: (fi[b], 0)),
            pl.BlockSpec((1, MINTER, HID),
                         lambda b, se, fi: (se[fi[b]], 0, 0)),
            pl.BlockSpec((1, HID, MINTER),
                         lambda b, se, fi: (se[fi[b]], 0, 0)),
        ],
        out_specs=pl.BlockSpec((BLK, HID), lambda b, se, fi: (b, 0)),
    )
    return pl.pallas_call(
        _expert_body,
        grid_spec=grid_spec,
        out_shape=jax.ShapeDtypeStruct((NS, HID), jnp.float32),
        compiler_params=pltpu.CompilerParams(
            dimension_semantics=("arbitrary",)),
    )(blk_e, fetch_i, xs, We1, We2)


def _shared_body(x_ref, ws1_ref, ws2_ref, o_ref):
    s = pl.program_id(1)

    @pl.when(s == 0)
    def _z():
        o_ref[...] = jnp.zeros_like(o_ref)

    x = x_ref[...]
    h = lax.dot_general(x, ws1_ref[...],
                        (((1,), (1,)), ((), ())),
                        preferred_element_type=jnp.float32)
    h = h * jax.nn.sigmoid(h)
    o_ref[...] += lax.dot_general(h,
                                  ws2_ref[...],
                                  (((1,), (1,)), ((), ())),
                                  preferred_element_type=jnp.float32)


def _shared_call(x, Ws1, Ws2):
    return pl.pallas_call(
        _shared_body,
        grid=(NCH, NSH),
        in_specs=[
            pl.BlockSpec((TT, HID), lambda t, s: (t, 0)),
            pl.BlockSpec((SH_IT, HID), lambda t, s: (s, 0)),
            pl.BlockSpec((HID, SH_IT), lambda t, s: (0, s)),
        ],
        out_specs=pl.BlockSpec((TT, HID), lambda t, s: (t, 0)),
        out_shape=jax.ShapeDtypeStruct((T, HID), jnp.float32),
        compiler_params=pltpu.CompilerParams(
            dimension_semantics=("parallel", "arbitrary")),
    )(x, Ws1, Ws2)


def _make_combine():
    mesh = plsc.VectorSubcoreMesh(core_axis_name="c", subcore_axis_name="s")

    @functools.partial(
        pl.kernel,
        out_type=jax.ShapeDtypeStruct((T, HID), jnp.float32),
        mesh=mesh,
        scratch_types=[
            pltpu.VMEM((TPW,), jnp.int32),
            pltpu.VMEM((TPW,), jnp.int32),
            pltpu.VMEM((TPW,), jnp.float32),
            pltpu.VMEM((TPW,), jnp.float32),
            pltpu.VMEM((2, RG2, HID), jnp.float32),
            pltpu.VMEM((2, RG2, HID), jnp.float32),
            pltpu.VMEM((2, RG2, HID), jnp.float32),
            pltpu.SemaphoreType.DMA,
            pltpu.SemaphoreType.DMA,
        ],
    )
    def combine(sh_hbm, y_hbm, inv0_hbm, inv1_hbm, w1_hbm, w2_hbm,
                out_hbm, idx0, idx1, w1v, w2v, acc, y0, y1, sem, osem):
        cc = lax.axis_index("c")
        ss = lax.axis_index("s")
        wid = ss * 2 + cc
        base = wid * TPW
        pltpu.sync_copy(inv0_hbm.at[pl.ds(base, TPW)], idx0)
        pltpu.sync_copy(inv1_hbm.at[pl.ds(base, TPW)], idx1)
        pltpu.sync_copy(w1_hbm.at[pl.ds(base, TPW)], w1v)
        pltpu.sync_copy(w2_hbm.at[pl.ds(base, TPW)], w2v)

        def fetch(g, buf):
            return [
                pltpu.async_copy(sh_hbm.at[pl.ds(base + g * RG2, RG2)],
                                 acc.at[buf], sem),
                pltpu.async_copy(y_hbm.at[idx0.at[pl.ds(g * RG2, RG2)]],
                                 y0.at[buf], sem),
                pltpu.async_copy(y_hbm.at[idx1.at[pl.ds(g * RG2, RG2)]],
                                 y1.at[buf], sem),
            ]

        pend = fetch(0, 0)
        outd = [None, None]
        for g in range(NG2):
            buf = g % 2
            for d in pend:
                d.wait()
            if g + 1 < NG2:
                pend = fetch(g + 1, (g + 1) % 2)
            if outd[buf] is not None:
                outd[buf].wait()
            w1g = w1v[pl.ds(g * RG2, 16)]
            w2g = w2v[pl.ds(g * RG2, 16)]
            for i in range(RG2):
                sel = jnp.full((16,), i, jnp.int32)
                wa = _take16(w1g, sel)
                wb = _take16(w2g, sel)

                def body(ci, carry):
                    sl = pl.ds(ci * 16, 16)
                    acc[buf, i, sl] = (acc[buf, i, sl]
                                       + wa * y0[buf, i, sl]
                                       + wb * y1[buf, i, sl])
                    return carry

                lax.fori_loop(0, HID // 16, body, 0)
            outd[buf] = pltpu.async_copy(
                acc.at[buf], out_hbm.at[pl.ds(base + g * RG2, RG2)], osem)
        for d in outd:
            if d is not None:
                d.wait()

    return combine


def kernel(hidden_states, Ws1, Ws2, Wg, We1, We2):
    B, S, H = hidden_states.shape
    x = hidden_states.reshape(T, H)

    (logits, i1, i2, w1, w2, r1, r2, cb, eb, be, fi) = _router_call(x, Wg)

    dispatch = _make_dispatch()
    xs, inv0, inv1 = dispatch(x, i1.reshape(T), i2.reshape(T),
                              r1.reshape(T), r2.reshape(T),
                              eb.reshape(NE), cb.reshape(NCH * NE))

    y = _expert_call(be, fi, xs, We1, We2)
    sh = _shared_call(x, Ws1, Ws2)

    combine = _make_combine()
    out = combine(sh, y, inv0, inv1, w1.reshape(T), w2.reshape(T))

    return out.reshape(B, S, H), logits.reshape(B, S, NE)
